# baseline (speedup 1.0000x reference)
import functools

import jax
import jax.numpy as jnp
from jax import lax
from jax.experimental import pallas as pl
from jax.experimental.pallas import tpu as pltpu
from jax.experimental.pallas import tpu_sc as plsc

M, K, D = 8, 8192, 256
N, H, W = 16, 32, 32

NC, NS = 2, 16
NW = NC * NS
LANES = 16

T = H * W
B = N * T * M
ROWS = B // 128
ROWS_PER_W = ROWS // NW
CHUNK = 128


def _sc_gather(table, codes2):
    mesh = plsc.VectorSubcoreMesh(
        core_axis_name="c", subcore_axis_name="s", num_cores=NC,
        num_subcores=NS)

    @functools.partial(
        pl.kernel,
        mesh=mesh,
        out_type=jax.ShapeDtypeStruct((B, D), jnp.float32),
        scratch_types=[
            pltpu.VMEM((ROWS_PER_W, 128), jnp.int32),
            pltpu.VMEM((CHUNK, D), jnp.float32),
            pltpu.SemaphoreType.DMA,
        ],
    )
    def k(table_hbm, codes_hbm, out_hbm, idx_v, rows_v, sem):
        wid = lax.axis_index("s") * NC + lax.axis_index("c")
        row0 = wid * ROWS_PER_W
        pltpu.sync_copy(codes_hbm.at[pl.ds(row0, ROWS_PER_W)], idx_v)

        mv = (lax.iota(jnp.int32, LANES) % M) * K

        def add_m(j, _):
            for c in range(128 // LANES):
                sl = pl.ds(c * LANES, LANES)
                idx_v[j, sl] = idx_v[j, sl] + mv
            return 0

        lax.fori_loop(0, ROWS_PER_W, add_m, 0)

        def gather_chunk(j, _):
            pltpu.async_copy(table_hbm.at[idx_v.at[j]], rows_v, sem).wait()
            pltpu.sync_copy(
                rows_v, out_hbm.at[pl.ds((row0 + j) * 128, CHUNK)])
            return 0

        lax.fori_loop(0, ROWS_PER_W, gather_chunk, 0)

    return k(table, codes2)


def _tc_transpose(g3):
    def body(in_ref, out_ref):
        out_ref[...] = jnp.swapaxes(in_ref[...], 1, 2)

    return pl.pallas_call(
        body,
        grid=(N, M * D // 128),
        in_specs=[pl.BlockSpec((1, T, 128), lambda n, c: (n, 0, c))],
        out_specs=pl.BlockSpec((1, 128, T), lambda n, c: (n, c, 0)),
        out_shape=jax.ShapeDtypeStruct((N, M * D, T), jnp.float32),
    )(g3)


def kernel(codes, codebook):
    table = codebook.reshape(M * K, D)
    codes2 = codes.reshape(ROWS, 128)
    g = _sc_gather(table, codes2)
    out = _tc_transpose(g.reshape(N, T, M * D))
    return out.reshape(N, M * D, H, W)


# speedup vs baseline: 1.1697x; 1.1697x over previous
import functools

import jax
import jax.numpy as jnp
from jax import lax
from jax.experimental import pallas as pl
from jax.experimental.pallas import tpu as pltpu
from jax.experimental.pallas import tpu_sc as plsc

M, K, D = 8, 8192, 256
N, H, W = 16, 32, 32

NC, NS = 2, 16
NW = NC * NS
LANES = 16

T = H * W
B = N * T * M
ROWS = B // 128
ROWS_PER_W = ROWS // NW
CHUNK = 128


def _sc_gather(table, codes2):
    n_rows = codes2.shape[0]
    rpw = n_rows // NW
    mesh = plsc.VectorSubcoreMesh(
        core_axis_name="c", subcore_axis_name="s", num_cores=NC,
        num_subcores=NS)

    @functools.partial(
        pl.kernel,
        mesh=mesh,
        out_type=jax.ShapeDtypeStruct((n_rows * 128, D), jnp.float32),
        scratch_types=[
            pltpu.VMEM((rpw, 128), jnp.int32),
            pltpu.VMEM((CHUNK, D), jnp.float32),
            pltpu.SemaphoreType.DMA,
        ],
    )
    def k(table_hbm, codes_hbm, out_hbm, idx_v, rows_v, sem):
        wid = lax.axis_index("s") * NC + lax.axis_index("c")
        row0 = wid * rpw
        pltpu.sync_copy(codes_hbm.at[pl.ds(row0, rpw)], idx_v)

        mv = (lax.iota(jnp.int32, LANES) % M) * K

        def add_m(j, _):
            for c in range(128 // LANES):
                sl = pl.ds(c * LANES, LANES)
                idx_v[j, sl] = idx_v[j, sl] + mv
            return 0

        lax.fori_loop(0, rpw, add_m, 0)

        def gather_chunk(j, _):
            pltpu.async_copy(table_hbm.at[idx_v.at[j]], rows_v, sem).wait()
            pltpu.sync_copy(
                rows_v, out_hbm.at[pl.ds((row0 + j) * 128, CHUNK)])
            return 0

        lax.fori_loop(0, rpw, gather_chunk, 0)

    return k(table, codes2)


NG = 4
GN = N // NG


def _tc_transpose_group(g3, out_prev, g):
    def body(in_ref, _, out_ref):
        out_ref[...] = jnp.swapaxes(in_ref[...], 1, 2)

    return pl.pallas_call(
        body,
        grid=(GN,),
        in_specs=[
            pl.BlockSpec((1, T, M * D), lambda n: (n, 0, 0)),
            pl.BlockSpec(memory_space=pltpu.MemorySpace.HBM),
        ],
        out_specs=pl.BlockSpec((1, M * D, T), lambda n, g=g: (g * GN + n, 0, 0)),
        out_shape=jax.ShapeDtypeStruct((N, M * D, T), jnp.float32),
        input_output_aliases={1: 0},
    )(g3, out_prev)


def kernel(codes, codebook):
    table = codebook.reshape(M * K, D)
    codes4 = codes.reshape(NG, ROWS // NG, 128)
    gs = [_sc_gather(table, codes4[g]).reshape(GN, T, M * D)
          for g in range(NG)]
    out = jnp.zeros((N, M * D, T), jnp.float32)
    for g in range(NG):
        out = _tc_transpose_group(gs[g], out, g)
    return out.reshape(N, M * D, H, W)


# speedup vs baseline: 1.2360x; 1.0567x over previous
import functools

import jax
import jax.numpy as jnp
from jax import lax
from jax.experimental import pallas as pl
from jax.experimental.pallas import tpu as pltpu
from jax.experimental.pallas import tpu_sc as plsc

M, K, D = 8, 8192, 256
N, H, W = 16, 32, 32

NC, NS = 2, 16
NW = NC * NS
LANES = 16

T = H * W
B = N * T * M
ROWS = B // 128
ROWS_PER_W = ROWS // NW
CHUNK = 128


def _sc_gather(table, codes2):
    mesh = plsc.VectorSubcoreMesh(
        core_axis_name="c", subcore_axis_name="s", num_cores=NC,
        num_subcores=NS)

    @functools.partial(
        pl.kernel,
        mesh=mesh,
        out_type=jax.ShapeDtypeStruct((B, D), jnp.float32),
        scratch_types=[
            pltpu.VMEM((ROWS_PER_W, 128), jnp.int32),
            pltpu.VMEM((CHUNK, D), jnp.float32),
            pltpu.SemaphoreType.DMA,
        ],
    )
    def k(table_hbm, codes_hbm, out_hbm, idx_v, rows_v, sem):
        wid = lax.axis_index("s") * NC + lax.axis_index("c")
        row0 = wid * ROWS_PER_W
        pltpu.sync_copy(codes_hbm.at[pl.ds(row0, ROWS_PER_W)], idx_v)

        mv = (lax.iota(jnp.int32, LANES) % M) * K

        def add_m(j, _):
            for c in range(128 // LANES):
                sl = pl.ds(c * LANES, LANES)
                idx_v[j, sl] = idx_v[j, sl] + mv
            return 0

        lax.fori_loop(0, ROWS_PER_W, add_m, 0)

        def gather_chunk(j, _):
            pltpu.async_copy(table_hbm.at[idx_v.at[j]], rows_v, sem).wait()
            pltpu.sync_copy(
                rows_v, out_hbm.at[pl.ds((row0 + j) * 128, CHUNK)])
            return 0

        lax.fori_loop(0, ROWS_PER_W, gather_chunk, 0)

    return k(table, codes2)


def _tc_transpose(g3):
    def body(in_ref, out_ref):
        out_ref[...] = jnp.swapaxes(in_ref[...], 1, 2)

    return pl.pallas_call(
        body,
        grid=(N,),
        in_specs=[pl.BlockSpec((1, T, M * D), lambda n: (n, 0, 0))],
        out_specs=pl.BlockSpec((1, M * D, T), lambda n: (n, 0, 0)),
        out_shape=jax.ShapeDtypeStruct((N, M * D, T), jnp.float32),
    )(g3)


def kernel(codes, codebook):
    table = codebook.reshape(M * K, D)
    codes2 = codes.reshape(ROWS, 128)
    g = _sc_gather(table, codes2)
    out = _tc_transpose(g.reshape(N, T, M * D))
    return out.reshape(N, M * D, H, W)
